# baseline (speedup 1.0000x reference)
import functools

import jax
import jax.numpy as jnp
from jax import lax
from jax.experimental import pallas as pl
from jax.experimental.pallas import tpu as pltpu
from jax.experimental.pallas import tpu_sc as plsc

H_UNITS = 32
H_SCALE = float(H_UNITS) ** 0.5

_NC = 2
_NS = 16
_NW = _NC * _NS

_B0 = 16384
_T = 26
_V = 1000000
_D = H_UNITS
_BW = _B0 // _NW
_RPG = 128
_NBLK = _BW // _RPG
_NK = _T * _NBLK


def _sc_body(idx_hbm, table_hbm, out_hbm, idx_v, glist, grows, out_v, gsem, ssem):
    wid = lax.axis_index("s") * _NC + lax.axis_index("c")
    bbase = wid * _BW

    pltpu.sync_copy(idx_hbm.at[:, pl.ds(bbase, _BW)], idx_v)

    iota16 = lax.iota(jnp.int32, 16)

    def build_glist(k):
        slot = k & 7
        t = lax.shift_right_logical(k, 2)
        blk = k & 3
        for u in range(8):
            vidx = idx_v[t, pl.ds(blk * _RPG + u * 16, 16)]
            glist[slot, pl.ds(u * 16, 16)] = lax.bitwise_or(
                lax.shift_left(lax.shift_right_logical(vidx, 5), 3),
                lax.bitwise_and(vidx, 7),
            )

    def fire_gather(k):
        slot = k & 7
        p = k & 1
        pltpu.async_copy(table_hbm.at[glist.at[slot]], grows.at[p], gsem)

    build_glist(0)
    fire_gather(0)

    def body(k, carry):
        p = k & 1
        t = lax.shift_right_logical(k, 2)
        blk = k & 3

        pltpu.make_async_copy(
            table_hbm.at[pl.ds(0, _RPG)], grows.at[p], gsem
        ).wait()

        @pl.when(k > 0)
        def _():
            pltpu.make_async_copy(
                out_v.at[0], out_hbm.at[0, :, pl.ds(bbase, _RPG)], ssem
            ).wait()

        @pl.when(k + 1 < _NK)
        def _():
            build_glist(k + 1)
            fire_gather(k + 1)

        src = grows.at[p]
        for u in range(8):
            vidx = idx_v[t, pl.ds(blk * _RPG + u * 16, 16)]
            colv = lax.shift_left(
                lax.bitwise_and(lax.shift_right_logical(vidx, 3), 3), 5
            )
            rowv = iota16 + (u * 16)

            @plsc.parallel_loop(0, _D, 1, unroll=8)
            def hloop(h, colv=colv, rowv=rowv, u=u):
                vals = plsc.load_gather(src, [rowv, colv + h])
                out_v[p, h, pl.ds(u * 16, 16)] = vals * H_SCALE

        pltpu.async_copy(
            out_v.at[p],
            out_hbm.at[t, :, pl.ds(bbase + blk * _RPG, _RPG)],
            ssem,
        )
        return carry

    lax.fori_loop(0, _NK, body, 0)

    pltpu.make_async_copy(
        out_v.at[0], out_hbm.at[0, :, pl.ds(bbase, _RPG)], ssem
    ).wait()


@jax.jit
def kernel(inputs, lookup_table):
    b0, t = inputs.shape
    assert (b0, t) == (_B0, _T) and lookup_table.shape == (_V, _D)
    idx_t = inputs.T.astype(jnp.int32)
    table2 = (
        lookup_table.reshape(_V // 32, 4, 8, _D)
        .transpose(0, 2, 1, 3)
        .reshape(_V * _D // 128, 128)
    )

    emb = pl.kernel(
        _sc_body,
        mesh=plsc.VectorSubcoreMesh(core_axis_name="c", subcore_axis_name="s"),
        out_type=jax.ShapeDtypeStruct((_T, _D, _B0), jnp.float32),
        compiler_params=pltpu.CompilerParams(
            use_tc_tiling_on_sc=True, needs_layout_passes=False
        ),
        scratch_types=[
            pltpu.VMEM((_T, _BW), jnp.int32),
            pltpu.VMEM((8, _RPG), jnp.int32),
            pltpu.VMEM((2, _RPG, 128), jnp.float32),
            pltpu.VMEM((2, _D, _RPG), jnp.float32),
            pltpu.SemaphoreType.DMA,
            pltpu.SemaphoreType.DMA,
        ],
    )
    out_t = emb(idx_t, table2)
    return jnp.transpose(out_t, (2, 0, 1))


# speedup vs baseline: 1.2946x; 1.2946x over previous
import functools

import jax
import jax.numpy as jnp
from jax import lax
from jax.experimental import pallas as pl
from jax.experimental.pallas import tpu as pltpu
from jax.experimental.pallas import tpu_sc as plsc

H_UNITS = 32
H_SCALE = float(H_UNITS) ** 0.5

_NC = 2
_NS = 16
_NW = _NC * _NS

_B0 = 16384
_T = 26
_V = 1000000
_D = H_UNITS
_BW = _B0 // _NW
_RPG = 128
_NBLK = _BW // _RPG
_NK = _T * _NBLK


def _sc_body(
    idx_hbm, table_hbm, out_hbm, idx_v, glist, grows, out_v, gsemA, gsemB, ssem
):
    wid = lax.axis_index("s") * _NC + lax.axis_index("c")
    bbase = wid * _BW

    pltpu.sync_copy(idx_hbm.at[:, pl.ds(bbase, _BW)], idx_v)

    iota16 = lax.iota(jnp.int32, 16)

    def build_glist(k):
        slot = k & 7
        t = lax.shift_right_logical(k, 2)
        blk = k & 3
        for u in range(8):
            vidx = idx_v[t, pl.ds(blk * _RPG + u * 16, 16)]
            glist[slot, pl.ds(u * 16, 16)] = lax.shift_right_logical(vidx, 2)

    def fire_gather(k, sem):
        slot = k & 7
        p = k & 1
        pltpu.async_copy(table_hbm.at[glist.at[slot]], grows.at[p], sem)

    def wait_gather(sem, p):
        pltpu.make_async_copy(
            table_hbm.at[pl.ds(0, _RPG)], grows.at[p], sem
        ).wait()

    build_glist(0)
    fire_gather(0, gsemA)
    build_glist(1)
    fire_gather(1, gsemB)

    def body(k, carry):
        p = k & 1
        t = lax.shift_right_logical(k, 2)
        blk = k & 3

        @pl.when(p == 0)
        def _():
            wait_gather(gsemA, p)

        @pl.when(p == 1)
        def _():
            wait_gather(gsemB, p)

        @pl.when(k > 0)
        def _():
            pltpu.make_async_copy(
                out_v.at[0], out_hbm.at[0, :, pl.ds(bbase, _RPG)], ssem
            ).wait()

        src = grows.at[p]
        for u in range(8):
            vidx = idx_v[t, pl.ds(blk * _RPG + u * 16, 16)]
            colv = lax.shift_left(lax.bitwise_and(vidx, 3), 5)
            rowv = iota16 + (u * 16)

            @plsc.parallel_loop(0, _D, 1, unroll=16)
            def hloop(h, colv=colv, rowv=rowv, u=u):
                vals = plsc.load_gather(src, [rowv, colv + h])
                out_v[p, h, pl.ds(u * 16, 16)] = vals * H_SCALE

        pltpu.async_copy(
            out_v.at[p],
            out_hbm.at[t, :, pl.ds(bbase + blk * _RPG, _RPG)],
            ssem,
        )

        @pl.when(jnp.logical_and(k + 2 < _NK, p == 0))
        def _():
            build_glist(k + 2)
            fire_gather(k + 2, gsemA)

        @pl.when(jnp.logical_and(k + 2 < _NK, p == 1))
        def _():
            build_glist(k + 2)
            fire_gather(k + 2, gsemB)

        return carry

    lax.fori_loop(0, _NK, body, 0)

    pltpu.make_async_copy(
        out_v.at[0], out_hbm.at[0, :, pl.ds(bbase, _RPG)], ssem
    ).wait()


@jax.jit
def kernel(inputs, lookup_table):
    b0, t = inputs.shape
    assert (b0, t) == (_B0, _T) and lookup_table.shape == (_V, _D)
    idx_t = inputs.T.astype(jnp.int32)
    table2 = lookup_table.reshape(_V * _D // 128, 128)

    emb = pl.kernel(
        _sc_body,
        mesh=plsc.VectorSubcoreMesh(core_axis_name="c", subcore_axis_name="s"),
        out_type=jax.ShapeDtypeStruct((_T, _D, _B0), jnp.float32),
        compiler_params=pltpu.CompilerParams(
            use_tc_tiling_on_sc=True, needs_layout_passes=False
        ),
        scratch_types=[
            pltpu.VMEM((_T, _BW), jnp.int32),
            pltpu.VMEM((8, _RPG), jnp.int32),
            pltpu.VMEM((2, _RPG, 128), jnp.float32),
            pltpu.VMEM((2, _D, _RPG), jnp.float32),
            pltpu.SemaphoreType.DMA,
            pltpu.SemaphoreType.DMA,
            pltpu.SemaphoreType.DMA,
        ],
    )
    out_t = emb(idx_t, table2)
    return jnp.transpose(out_t, (2, 0, 1))


# speedup vs baseline: 1.3010x; 1.0049x over previous
import jax
import jax.numpy as jnp
from jax import lax
from jax.experimental import pallas as pl
from jax.experimental.pallas import tpu as pltpu
from jax.experimental.pallas import tpu_sc as plsc

H_UNITS = 32
H_SCALE = float(H_UNITS) ** 0.5

_NC = 2
_NS = 16
_NW = _NC * _NS

_B0 = 16384
_T = 26
_V = 1000000
_D = H_UNITS
_BW = _B0 // _NW
_RPG = 128
_NBLK = _BW // _RPG
_NK = _T * _NBLK


def _sc_body(idx_hbm, table_hbm, out_hbm, idx_v, glist, grows, out_v, gsem, ssem):
    wid = lax.axis_index("s") * _NC + lax.axis_index("c")
    bbase = wid * _BW

    pltpu.sync_copy(idx_hbm.at[:, pl.ds(bbase, _BW)], idx_v)

    iota16 = lax.iota(jnp.int32, 16)

    def build_glist(k):
        slot = k & 7
        t = lax.shift_right_logical(k, 2)
        blk = k & 3
        for u in range(8):
            vidx = idx_v[t, pl.ds(blk * _RPG + u * 16, 16)]
            glist[slot, pl.ds(u * 16, 16)] = lax.shift_right_logical(vidx, 2)

    def fire_gather(k):
        slot = k & 7
        p = k & 1
        pltpu.async_copy(table_hbm.at[glist.at[slot]], grows.at[p], gsem)

    build_glist(0)
    fire_gather(0)

    def body(k, carry):
        p = k & 1
        t = lax.shift_right_logical(k, 2)
        blk = k & 3

        pltpu.make_async_copy(
            table_hbm.at[pl.ds(0, _RPG)], grows.at[p], gsem
        ).wait()

        @pl.when(k > 0)
        def _():
            pltpu.make_async_copy(
                out_v.at[0], out_hbm.at[0, :, pl.ds(bbase, _RPG)], ssem
            ).wait()

        @pl.when(k + 1 < _NK)
        def _():
            build_glist(k + 1)
            fire_gather(k + 1)

        src = grows.at[p]
        for u in range(8):
            vidx = idx_v[t, pl.ds(blk * _RPG + u * 16, 16)]
            colv = lax.shift_left(lax.bitwise_and(vidx, 3), 5)
            rowv = iota16 + (u * 16)

            @plsc.parallel_loop(0, _D, 1, unroll=8)
            def hloop(h, colv=colv, rowv=rowv, u=u):
                vals = plsc.load_gather(src, [rowv, colv + h])
                out_v[p, h, pl.ds(u * 16, 16)] = vals * H_SCALE

        pltpu.async_copy(
            out_v.at[p],
            out_hbm.at[t, :, pl.ds(bbase + blk * _RPG, _RPG)],
            ssem,
        )
        return carry

    lax.fori_loop(0, _NK, body, 0)

    pltpu.make_async_copy(
        out_v.at[0], out_hbm.at[0, :, pl.ds(bbase, _RPG)], ssem
    ).wait()


@jax.jit
def kernel(inputs, lookup_table):
    b0, t = inputs.shape
    assert (b0, t) == (_B0, _T) and lookup_table.shape == (_V, _D)
    idx_t = inputs.T.astype(jnp.int32)
    table2 = lookup_table.reshape(_V * _D // 128, 128)

    emb = pl.kernel(
        _sc_body,
        mesh=plsc.VectorSubcoreMesh(core_axis_name="c", subcore_axis_name="s"),
        out_type=jax.ShapeDtypeStruct((_T, _D, _B0), jnp.float32),
        compiler_params=pltpu.CompilerParams(
            use_tc_tiling_on_sc=True, needs_layout_passes=False
        ),
        scratch_types=[
            pltpu.VMEM((_T, _BW), jnp.int32),
            pltpu.VMEM((8, _RPG), jnp.int32),
            pltpu.VMEM((2, _RPG, 128), jnp.float32),
            pltpu.VMEM((2, _D, _RPG), jnp.float32),
            pltpu.SemaphoreType.DMA,
            pltpu.SemaphoreType.DMA,
        ],
    )
    out_t = emb(idx_t, table2)
    return jnp.transpose(out_t, (2, 0, 1))
